# baseline (speedup 1.0000x reference)
import functools

import jax
import jax.numpy as jnp
from jax import lax
from jax.experimental import pallas as pl
from jax.experimental.pallas import tpu as pltpu

SELU_SCALE = 1.0507009873554804934193349852946


def _gate_kernel(zp_ref, w1sq_ref, b1sq_ref, w1ex_ref, b1ex_ref,
                 w2sq_ref, b2sq_ref, w2ex_ref, b2ex_ref,
                 gamma_ref, beta_ref,
                 *, nb, seg, wp, wz, l2, inv_count, c_out):
    offs = [(t // 3) * wp + (t % 3) for t in range(9)]
    l1_full = nb * seg - offs[-1]
    l2_full = l1_full - offs[-1]

    s1 = jnp.dot(w1sq_ref[...], zp_ref[0],
                 preferred_element_type=jnp.float32) + b1sq_ref[...]
    s1 = jnp.maximum(s1, 0.0).astype(jnp.bfloat16)

    r1 = jnp.concatenate([s1[:, o:o + l1_full] for o in offs], axis=0)
    y1 = jnp.dot(w1ex_ref[...], r1,
                 preferred_element_type=jnp.float32) + b1ex_ref[...]
    y1 = (SELU_SCALE * jnp.maximum(y1, 0.0)).astype(jnp.bfloat16)

    s2 = jnp.dot(w2sq_ref[...], y1,
                 preferred_element_type=jnp.float32) + b2sq_ref[...]
    s2 = jnp.maximum(s2, 0.0).astype(jnp.bfloat16)

    r2 = jnp.concatenate([s2[:, o:o + l2_full] for o in offs], axis=0)
    g = jnp.dot(w2ex_ref[...], r2,
                preferred_element_type=jnp.float32) + b2ex_ref[...]
    g = jnp.maximum(g, 0.0)

    col = lax.broadcasted_iota(jnp.int32, (1, l2), 1)
    mask = (lax.rem(col, wp) < wz).astype(jnp.float32)
    for b in range(nb):
        gb = g[:, b * seg:b * seg + l2]
        means = jnp.sum(gb * mask, axis=-1, keepdims=True) * inv_count
        gamma_ref[b] = 3.0 / (1.0 + jnp.exp(-means[:c_out]))
        beta_ref[b] = means[c_out:]


def _adain_kernel(x_ref, gamma_ref, beta_ref, o_ref, *, inv_n):
    x = x_ref[0]
    mu = jnp.sum(x, axis=-1, keepdims=True) * inv_n
    ex2 = jnp.sum(x * x, axis=-1, keepdims=True) * inv_n
    var = ex2 - mu * mu
    inv = lax.rsqrt(var + 1e-5)
    o_ref[0] = (x - mu) * (inv * gamma_ref[0]) + beta_ref[0]


def kernel(x, z, f1_sq_w, f1_sq_b, f1_ex_w, f1_ex_b,
           f2_sq_w, f2_sq_b, f2_ex_w, f2_ex_b):
    n, cz, hz, wz = z.shape
    _, c, h, w = x.shape
    hp, wp = hz + 4, wz + 4
    seg = hp * wp
    l2 = hz * wp - 4
    cx = f2_ex_w.shape[1] // 2

    zp = jnp.pad(z, ((0, 0), (0, 0), (2, 2), (2, 2)), mode="reflect")
    zp = jnp.transpose(zp.reshape(n, cz, seg), (1, 0, 2))
    zp = zp.reshape(cz, n * seg).astype(jnp.bfloat16)
    ngrid = 2 if n % 2 == 0 else 1
    nb = n // ngrid
    blk = nb * seg
    zp = jnp.transpose(zp.reshape(cz, ngrid, blk), (1, 0, 2))

    c1, sq1 = f1_ex_w.shape[1], f1_ex_w.shape[2]
    c2, sq2 = f2_ex_w.shape[1], f2_ex_w.shape[2]
    w1ex = jnp.transpose(f1_ex_w, (1, 0, 2)).reshape(c1, 9 * sq1)
    w2ex = jnp.transpose(f2_ex_w, (1, 0, 2)).reshape(c2, 9 * sq2)

    gate_fn = functools.partial(
        _gate_kernel, nb=nb, seg=seg, wp=wp, wz=wz, l2=l2,
        inv_count=1.0 / float(hz * wz), c_out=cx)
    gamma, beta = pl.pallas_call(
        gate_fn,
        out_shape=(jax.ShapeDtypeStruct((n, cx, 1), jnp.float32),
                   jax.ShapeDtypeStruct((n, cx, 1), jnp.float32)),
        grid=(ngrid,),
        in_specs=[
            pl.BlockSpec((1, cz, blk), lambda g: (g, 0, 0)),
            pl.BlockSpec(f1_sq_w.shape, lambda g: (0, 0)),
            pl.BlockSpec(f1_sq_b.shape, lambda g: (0, 0)),
            pl.BlockSpec((c1, 9 * sq1), lambda g: (0, 0)),
            pl.BlockSpec(f1_ex_b.shape, lambda g: (0, 0)),
            pl.BlockSpec(f2_sq_w.shape, lambda g: (0, 0)),
            pl.BlockSpec(f2_sq_b.shape, lambda g: (0, 0)),
            pl.BlockSpec((c2, 9 * sq2), lambda g: (0, 0)),
            pl.BlockSpec(f2_ex_b.shape, lambda g: (0, 0)),
        ],
        out_specs=(pl.BlockSpec((nb, cx, 1), lambda g: (g, 0, 0)),
                   pl.BlockSpec((nb, cx, 1), lambda g: (g, 0, 0))),
        compiler_params=pltpu.CompilerParams(dimension_semantics=("parallel",)),
    )(zp, f1_sq_w.astype(jnp.bfloat16), f1_sq_b,
      w1ex.astype(jnp.bfloat16), f1_ex_b,
      f2_sq_w.astype(jnp.bfloat16), f2_sq_b,
      w2ex.astype(jnp.bfloat16), f2_ex_b)

    hw = h * w
    ct = 64 if c % 64 == 0 else c
    adain_fn = functools.partial(_adain_kernel, inv_n=1.0 / float(hw))
    out = pl.pallas_call(
        adain_fn,
        out_shape=jax.ShapeDtypeStruct((n, c, hw), x.dtype),
        grid=(n, c // ct),
        in_specs=[pl.BlockSpec((1, ct, hw), lambda b, i: (b, i, 0)),
                  pl.BlockSpec((1, ct, 1), lambda b, i: (b, i, 0)),
                  pl.BlockSpec((1, ct, 1), lambda b, i: (b, i, 0))],
        out_specs=pl.BlockSpec((1, ct, hw), lambda b, i: (b, i, 0)),
        compiler_params=pltpu.CompilerParams(
            dimension_semantics=("parallel", "parallel")),
    )(x.reshape(n, c, hw), gamma, beta)
    return out.reshape(n, c, h, w)


# speedup vs baseline: 1.1463x; 1.1463x over previous
import functools

import jax
import jax.numpy as jnp
from jax import lax
from jax.experimental import pallas as pl
from jax.experimental.pallas import tpu as pltpu

SELU_SCALE = 1.0507009873554804934193349852946


def _fused_kernel(zp_ref, x_ref,
                  w1sq_ref, b1sq_ref, w1ex_ref, b1ex_ref,
                  w2sq_ref, b2sq_ref, w2ex_ref, b2ex_ref,
                  o_ref, *, wp, wz, l2, inv_count, c_out, inv_n):
    offs = [(t // 3) * wp + (t % 3) for t in range(9)]
    seg = zp_ref.shape[-1]
    l1 = seg - offs[-1]

    s1 = jnp.dot(w1sq_ref[...], zp_ref[0],
                 preferred_element_type=jnp.float32) + b1sq_ref[...]
    s1 = jnp.maximum(s1, 0.0).astype(jnp.bfloat16)

    r1 = jnp.concatenate([s1[:, o:o + l1] for o in offs], axis=0)
    y1 = jnp.dot(w1ex_ref[...], r1,
                 preferred_element_type=jnp.float32) + b1ex_ref[...]
    y1 = (SELU_SCALE * jnp.maximum(y1, 0.0)).astype(jnp.bfloat16)

    s2 = jnp.dot(w2sq_ref[...], y1,
                 preferred_element_type=jnp.float32) + b2sq_ref[...]
    s2 = jnp.maximum(s2, 0.0).astype(jnp.bfloat16)

    r2 = jnp.concatenate([s2[:, o:o + l2] for o in offs], axis=0)
    g = jnp.dot(w2ex_ref[...], r2,
                preferred_element_type=jnp.float32) + b2ex_ref[...]
    g = jnp.maximum(g, 0.0)

    col = lax.broadcasted_iota(jnp.int32, (1, l2), 1)
    mask = (lax.rem(col, wp) < wz).astype(jnp.float32)
    means = jnp.sum(g * mask, axis=-1, keepdims=True) * inv_count
    gamma = 3.0 / (1.0 + jnp.exp(-means[:c_out]))
    beta = means[c_out:]

    x = x_ref[0]
    mu = jnp.sum(x, axis=-1, keepdims=True) * inv_n
    ex2 = jnp.sum(x * x, axis=-1, keepdims=True) * inv_n
    inv = lax.rsqrt(ex2 - mu * mu + 1e-5)
    o_ref[0] = (x - mu) * (inv * gamma) + beta


def kernel(x, z, f1_sq_w, f1_sq_b, f1_ex_w, f1_ex_b,
           f2_sq_w, f2_sq_b, f2_ex_w, f2_ex_b):
    n, cz, hz, wz = z.shape
    _, c, h, w = x.shape
    hp, wp = hz + 4, wz + 4
    seg = hp * wp
    l2 = hz * wp - 4
    cx = f2_ex_w.shape[1] // 2
    hw = h * w

    zp = jnp.pad(z, ((0, 0), (0, 0), (2, 2), (2, 2)), mode="reflect")
    zp = zp.reshape(n, cz, seg).astype(jnp.bfloat16)

    c1, sq1 = f1_ex_w.shape[1], f1_ex_w.shape[2]
    c2, sq2 = f2_ex_w.shape[1], f2_ex_w.shape[2]
    w1ex = jnp.transpose(f1_ex_w.astype(jnp.bfloat16), (1, 0, 2))
    w2ex = jnp.transpose(f2_ex_w.astype(jnp.bfloat16), (1, 0, 2))

    fn = functools.partial(
        _fused_kernel, wp=wp, wz=wz, l2=l2,
        inv_count=1.0 / float(hz * wz), c_out=cx, inv_n=1.0 / float(hw))
    out = pl.pallas_call(
        fn,
        out_shape=jax.ShapeDtypeStruct((n, c, hw), x.dtype),
        grid=(n,),
        in_specs=[
            pl.BlockSpec((1, cz, seg), lambda b: (b, 0, 0)),
            pl.BlockSpec((1, c, hw), lambda b: (b, 0, 0)),
            pl.BlockSpec(f1_sq_w.shape, lambda b: (0, 0)),
            pl.BlockSpec(f1_sq_b.shape, lambda b: (0, 0)),
            pl.BlockSpec((c1, 9 * sq1), lambda b: (0, 0)),
            pl.BlockSpec(f1_ex_b.shape, lambda b: (0, 0)),
            pl.BlockSpec(f2_sq_w.shape, lambda b: (0, 0)),
            pl.BlockSpec(f2_sq_b.shape, lambda b: (0, 0)),
            pl.BlockSpec((c2, 9 * sq2), lambda b: (0, 0)),
            pl.BlockSpec(f2_ex_b.shape, lambda b: (0, 0)),
        ],
        out_specs=pl.BlockSpec((1, c, hw), lambda b: (b, 0, 0)),
        compiler_params=pltpu.CompilerParams(
            dimension_semantics=("arbitrary",)),
    )(zp, x.reshape(n, c, hw),
      f1_sq_w.astype(jnp.bfloat16), f1_sq_b,
      w1ex.reshape(c1, 9 * sq1), f1_ex_b,
      f2_sq_w.astype(jnp.bfloat16), f2_sq_b,
      w2ex.reshape(c2, 9 * sq2), f2_ex_b)
    return out.reshape(n, c, h, w)


# speedup vs baseline: 1.1709x; 1.0215x over previous
import functools

import jax
import jax.numpy as jnp
from jax import lax
from jax.experimental import pallas as pl
from jax.experimental.pallas import tpu as pltpu

SELU_SCALE = 1.0507009873554804934193349852946


def _fused_kernel(zp_ref, x_ref,
                  w1sq_ref, b1sq_ref, w1ex_ref, b1ex_ref,
                  w2sq_ref, b2sq_ref, w2ex_ref, b2ex_ref,
                  o_ref, *, wp, wz, l2, inv_count, c_out, inv_n):
    offs = [(t // 3) * wp + (t % 3) for t in range(9)]
    seg = zp_ref.shape[-1]
    l1 = seg - offs[-1]
    c1 = w1ex_ref.shape[1]
    c2 = w2ex_ref.shape[1]

    s1 = jnp.dot(w1sq_ref[...], zp_ref[0],
                 preferred_element_type=jnp.float32) + b1sq_ref[...]
    s1 = jnp.maximum(s1, 0.0).astype(jnp.bfloat16)

    acc = jnp.zeros((c1, l1), jnp.float32)
    for t in range(9):
        acc = acc + jnp.dot(w1ex_ref[t], s1[:, offs[t]:offs[t] + l1],
                            preferred_element_type=jnp.float32)
    y1 = (SELU_SCALE * jnp.maximum(acc + b1ex_ref[...], 0.0)).astype(jnp.bfloat16)

    s2 = jnp.dot(w2sq_ref[...], y1,
                 preferred_element_type=jnp.float32) + b2sq_ref[...]
    s2 = jnp.maximum(s2, 0.0).astype(jnp.bfloat16)

    acc2 = jnp.zeros((c2, l2), jnp.float32)
    for t in range(9):
        acc2 = acc2 + jnp.dot(w2ex_ref[t], s2[:, offs[t]:offs[t] + l2],
                              preferred_element_type=jnp.float32)
    g = jnp.maximum(acc2 + b2ex_ref[...], 0.0)

    col = lax.broadcasted_iota(jnp.int32, (1, l2), 1)
    mask = (lax.rem(col, wp) < wz).astype(jnp.float32)
    means = jnp.sum(g * mask, axis=-1, keepdims=True) * inv_count
    gamma = 3.0 / (1.0 + jnp.exp(-means[:c_out]))
    beta = means[c_out:]

    x = x_ref[0]
    mu = jnp.sum(x, axis=-1, keepdims=True) * inv_n
    ex2 = jnp.sum(x * x, axis=-1, keepdims=True) * inv_n
    inv = lax.rsqrt(ex2 - mu * mu + 1e-5)
    o_ref[0] = (x - mu) * (inv * gamma) + beta


def kernel(x, z, f1_sq_w, f1_sq_b, f1_ex_w, f1_ex_b,
           f2_sq_w, f2_sq_b, f2_ex_w, f2_ex_b):
    n, cz, hz, wz = z.shape
    _, c, h, w = x.shape
    hp, wp = hz + 4, wz + 4
    seg = hp * wp
    l2 = hz * wp - 4
    cx = f2_ex_w.shape[1] // 2
    hw = h * w

    zp = jnp.pad(z, ((0, 0), (0, 0), (2, 2), (2, 2)), mode="reflect")
    zp = zp.reshape(n, cz, seg).astype(jnp.bfloat16)

    fn = functools.partial(
        _fused_kernel, wp=wp, wz=wz, l2=l2,
        inv_count=1.0 / float(hz * wz), c_out=cx, inv_n=1.0 / float(hw))
    out = pl.pallas_call(
        fn,
        out_shape=jax.ShapeDtypeStruct((n, c, hw), x.dtype),
        grid=(n,),
        in_specs=[
            pl.BlockSpec((1, cz, seg), lambda b: (b, 0, 0)),
            pl.BlockSpec((1, c, hw), lambda b: (b, 0, 0)),
            pl.BlockSpec(f1_sq_w.shape, lambda b: (0, 0)),
            pl.BlockSpec(f1_sq_b.shape, lambda b: (0, 0)),
            pl.BlockSpec(f1_ex_w.shape, lambda b: (0, 0, 0)),
            pl.BlockSpec(f1_ex_b.shape, lambda b: (0, 0)),
            pl.BlockSpec(f2_sq_w.shape, lambda b: (0, 0)),
            pl.BlockSpec(f2_sq_b.shape, lambda b: (0, 0)),
            pl.BlockSpec(f2_ex_w.shape, lambda b: (0, 0, 0)),
            pl.BlockSpec(f2_ex_b.shape, lambda b: (0, 0)),
        ],
        out_specs=pl.BlockSpec((1, c, hw), lambda b: (b, 0, 0)),
        compiler_params=pltpu.CompilerParams(
            dimension_semantics=("arbitrary",)),
    )(zp, x.reshape(n, c, hw),
      f1_sq_w.astype(jnp.bfloat16), f1_sq_b,
      f1_ex_w.astype(jnp.bfloat16), f1_ex_b,
      f2_sq_w.astype(jnp.bfloat16), f2_sq_b,
      f2_ex_w.astype(jnp.bfloat16), f2_ex_b)
    return out.reshape(n, c, h, w)


# speedup vs baseline: 1.2521x; 1.0693x over previous
import functools

import jax
import jax.numpy as jnp
from jax import lax
from jax.experimental import pallas as pl
from jax.experimental.pallas import tpu as pltpu

SELU_SCALE = 1.0507009873554804934193349852946


def _fused_kernel(zp_ref, x_ref,
                  w1sq_ref, b1sq_ref, w1ex_ref, b1ex_ref,
                  w2sq_ref, b2sq_ref, w2ex_ref, b2ex_ref,
                  o_ref,
                  w1sq_s, w1ex_s, w2sq_s, w2ex_s,
                  *, wp, wz, l2, inv_count, c_out, inv_n):
    offs = [(t // 3) * wp + (t % 3) for t in range(9)]
    seg = zp_ref.shape[-1]
    l1 = seg - offs[-1]
    c1 = w1ex_ref.shape[1]
    c2 = w2ex_ref.shape[1]

    @pl.when(pl.program_id(0) == 0)
    def _():
        w1sq_s[...] = w1sq_ref[...].astype(jnp.bfloat16)
        w1ex_s[...] = w1ex_ref[...].astype(jnp.bfloat16)
        w2sq_s[...] = w2sq_ref[...].astype(jnp.bfloat16)
        w2ex_s[...] = w2ex_ref[...].astype(jnp.bfloat16)

    s1 = jnp.dot(w1sq_s[...], zp_ref[0],
                 preferred_element_type=jnp.float32) + b1sq_ref[...]
    s1 = jnp.maximum(s1, 0.0).astype(jnp.bfloat16)

    acc = jnp.zeros((c1, l1), jnp.float32)
    for t in range(9):
        acc = acc + jnp.dot(w1ex_s[t], s1[:, offs[t]:offs[t] + l1],
                            preferred_element_type=jnp.float32)
    y1 = (SELU_SCALE * jnp.maximum(acc + b1ex_ref[...], 0.0)).astype(jnp.bfloat16)

    s2 = jnp.dot(w2sq_s[...], y1,
                 preferred_element_type=jnp.float32) + b2sq_ref[...]
    s2 = jnp.maximum(s2, 0.0).astype(jnp.bfloat16)

    acc2 = jnp.zeros((c2, l2), jnp.float32)
    for t in range(9):
        acc2 = acc2 + jnp.dot(w2ex_s[t], s2[:, offs[t]:offs[t] + l2],
                              preferred_element_type=jnp.float32)
    g = jnp.maximum(acc2 + b2ex_ref[...], 0.0)

    col = lax.broadcasted_iota(jnp.int32, (1, l2), 1)
    mask = (lax.rem(col, wp) < wz).astype(jnp.float32)
    means = jnp.sum(g * mask, axis=-1, keepdims=True) * inv_count
    gamma = 3.0 / (1.0 + jnp.exp(-means[:c_out]))
    beta = means[c_out:]

    x = x_ref[0]
    mu = jnp.sum(x, axis=-1, keepdims=True) * inv_n
    ex2 = jnp.sum(x * x, axis=-1, keepdims=True) * inv_n
    inv = lax.rsqrt(ex2 - mu * mu + 1e-5)
    o_ref[0] = (x - mu) * (inv * gamma) + beta


def kernel(x, z, f1_sq_w, f1_sq_b, f1_ex_w, f1_ex_b,
           f2_sq_w, f2_sq_b, f2_ex_w, f2_ex_b):
    n, cz, hz, wz = z.shape
    _, c, h, w = x.shape
    hp, wp = hz + 4, wz + 4
    seg = hp * wp
    l2 = hz * wp - 4
    cx = f2_ex_w.shape[1] // 2
    hw = h * w

    zp = jnp.pad(z, ((0, 0), (0, 0), (2, 2), (2, 2)), mode="reflect")
    zp = zp.reshape(n, cz, seg).astype(jnp.bfloat16)

    csq1 = f1_sq_w.shape[0]
    csq2 = f2_sq_w.shape[0]
    c1 = f1_ex_w.shape[1]
    c2 = f2_ex_w.shape[1]

    fn = functools.partial(
        _fused_kernel, wp=wp, wz=wz, l2=l2,
        inv_count=1.0 / float(hz * wz), c_out=cx, inv_n=1.0 / float(hw))
    out = pl.pallas_call(
        fn,
        out_shape=jax.ShapeDtypeStruct((n, c, hw), x.dtype),
        grid=(n,),
        in_specs=[
            pl.BlockSpec((1, cz, seg), lambda b: (b, 0, 0)),
            pl.BlockSpec((1, c, hw), lambda b: (b, 0, 0)),
            pl.BlockSpec(f1_sq_w.shape, lambda b: (0, 0)),
            pl.BlockSpec(f1_sq_b.shape, lambda b: (0, 0)),
            pl.BlockSpec(f1_ex_w.shape, lambda b: (0, 0, 0)),
            pl.BlockSpec(f1_ex_b.shape, lambda b: (0, 0)),
            pl.BlockSpec(f2_sq_w.shape, lambda b: (0, 0)),
            pl.BlockSpec(f2_sq_b.shape, lambda b: (0, 0)),
            pl.BlockSpec(f2_ex_w.shape, lambda b: (0, 0, 0)),
            pl.BlockSpec(f2_ex_b.shape, lambda b: (0, 0)),
        ],
        out_specs=pl.BlockSpec((1, c, hw), lambda b: (b, 0, 0)),
        scratch_shapes=[pltpu.VMEM((csq1, cz), jnp.bfloat16),
                        pltpu.VMEM((9, c1, csq1), jnp.bfloat16),
                        pltpu.VMEM((csq2, c1), jnp.bfloat16),
                        pltpu.VMEM((9, c2, csq2), jnp.bfloat16)],
        compiler_params=pltpu.CompilerParams(
            dimension_semantics=("arbitrary",)),
    )(zp, x.reshape(n, c, hw),
      f1_sq_w, f1_sq_b, f1_ex_w, f1_ex_b,
      f2_sq_w, f2_sq_b, f2_ex_w, f2_ex_b)
    return out.reshape(n, c, h, w)
